# Initial kernel scaffold; baseline (speedup 1.0000x reference)
import jax
import jax.numpy as jnp
from jax.experimental import pallas as pl


def kernel(x, edge_index):
    raise NotImplementedError("write your pallas kernel here")


# speedup vs baseline: 7.8560x; 7.8560x over previous
import functools

import jax
import jax.numpy as jnp
from jax import lax
from jax.experimental import pallas as pl
from jax.experimental.pallas import tpu as pltpu
from jax.experimental.pallas import tpu_sc as plsc

_NC = 2
_NS = 16
_CE = 80


def _ew1_body(x_ref, h_ref, xt_ref):
    xt = jnp.tanh(x_ref[...])
    xt_ref[...] = xt
    h_ref[...] = jnp.sign(xt)


def _scale1_body(deg_ref, xt_ref, dinv_ref, z_ref):
    dinv = lax.rsqrt(deg_ref[...] + 1.0)
    dinv_ref[...] = dinv
    z_ref[...] = dinv * xt_ref[...]


def _mid_body(dinv_ref, s1_ref, z_ref, z2_ref):
    dinv = dinv_ref[...]
    z2_ref[...] = dinv * dinv * (s1_ref[...] + z_ref[...])


def _fin_body(dinv_ref, s2_ref, z2_ref, h2_ref):
    h2_ref[...] = dinv_ref[...] * (s2_ref[...] + z2_ref[...])


def _row_spec(br, d):
    return pl.BlockSpec((br, d), lambda i: (i, 0))


def _ew1(x):
    n, d = x.shape
    br = 2000
    return pl.pallas_call(
        _ew1_body,
        grid=(n // br,),
        in_specs=[_row_spec(br, d)],
        out_specs=[_row_spec(br, d), _row_spec(br, d)],
        out_shape=[jax.ShapeDtypeStruct((n, d), jnp.float32)] * 2,
    )(x)


def _scale1(degc, xt):
    n, d = xt.shape
    br = 2000
    return pl.pallas_call(
        _scale1_body,
        grid=(n // br,),
        in_specs=[_row_spec(br, 1), _row_spec(br, d)],
        out_specs=[_row_spec(br, 1), _row_spec(br, d)],
        out_shape=[jax.ShapeDtypeStruct((n, 1), jnp.float32),
                   jax.ShapeDtypeStruct((n, d), jnp.float32)],
    )(degc, xt)


def _mid(dinv, s1, z):
    n, d = z.shape
    br = 2000
    return pl.pallas_call(
        _mid_body,
        grid=(n // br,),
        in_specs=[_row_spec(br, 1), _row_spec(br, d), _row_spec(br, d)],
        out_specs=_row_spec(br, d),
        out_shape=jax.ShapeDtypeStruct((n, d), jnp.float32),
    )(dinv, s1, z)


def _fin(dinv, s2, z2):
    n, d = z2.shape
    br = 2000
    return pl.pallas_call(
        _fin_body,
        grid=(n // br,),
        in_specs=[_row_spec(br, 1), _row_spec(br, d), _row_spec(br, d)],
        out_specs=_row_spec(br, d),
        out_shape=jax.ShapeDtypeStruct((n, d), jnp.float32),
    )(dinv, s2, z2)


@functools.lru_cache(maxsize=None)
def _make_deg(n, e):
    half = n // _NC
    ept = e // _NS
    nchunk = ept // _CE
    dc = 1000
    ndc = half // dc
    kd = -(-ndc // _NS)
    mesh = plsc.VectorSubcoreMesh(core_axis_name="c", subcore_axis_name="s")

    @functools.partial(
        pl.kernel,
        out_type=jax.ShapeDtypeStruct((n,), jnp.float32),
        mesh=mesh,
        scratch_types=[
            pltpu.VMEM((_CE,), jnp.int32),
            pltpu.VMEM((1, _CE), jnp.int32),
            pltpu.VMEM((_CE,), jnp.float32),
            pltpu.VMEM((dc,), jnp.float32),
            pltpu.VMEM_SHARED((half + 8,), jnp.float32),
        ],
        compiler_params=pltpu.CompilerParams(use_tc_tiling_on_sc=False),
    )
    def deg_k(dst_hbm, out_hbm, dbuf, widx, ones, stage, acc):
        c = lax.axis_index("c")
        s = lax.axis_index("s")
        base_node = c * half
        for v in range(_CE // 16):
            ones[pl.ds(v * 16, 16)] = jnp.ones((16,), jnp.float32)
        zeros16 = jnp.zeros((16,), jnp.float32)

        def zstage(v, carry):
            stage[pl.ds(v * 16, 16)] = zeros16
            return carry
        lax.fori_loop(0, dc // 16, zstage, None)

        def zacc(k, carry):
            cc = s + k * _NS

            @pl.when(cc < ndc)
            def _():
                pltpu.sync_copy(stage, acc.at[pl.ds(cc * dc, dc)])
            return carry
        lax.fori_loop(0, kd, zacc, None)

        @pl.when(s == 0)
        def _():
            pltpu.sync_copy(stage.at[pl.ds(0, 8)], acc.at[pl.ds(half, 8)])
        plsc.subcore_barrier()

        ebase = s * ept

        def ebody(j, carry):
            pltpu.sync_copy(dst_hbm.at[pl.ds(ebase + j * _CE, _CE)], dbuf)
            for v in range(_CE // 16):
                d = dbuf[pl.ds(v * 16, 16)]
                local = d - base_node
                ok = (local >= 0) & (local < half)
                widx[0, pl.ds(v * 16, 16)] = jnp.where(ok, local, half)
            pltpu.sync_copy(ones, acc.at[widx.at[0]], add=True)
            return carry
        lax.fori_loop(0, nchunk, ebody, None)
        plsc.subcore_barrier()

        def dbody(k, carry):
            cc = s + k * _NS

            @pl.when(cc < ndc)
            def _():
                pltpu.sync_copy(acc.at[pl.ds(cc * dc, dc)], stage)
                pltpu.sync_copy(
                    stage, out_hbm.at[pl.ds(base_node + cc * dc, dc)])
            return carry
        lax.fori_loop(0, kd, dbody, None)

    return deg_k


@functools.lru_cache(maxsize=None)
def _make_prop(n, d, e):
    half = n // _NC
    ept = e // _NS
    nchunk = ept // _CE
    rc = 200
    nrc = half // rc
    kr = -(-nrc // _NS)
    mesh = plsc.VectorSubcoreMesh(core_axis_name="c", subcore_axis_name="s")

    @functools.partial(
        pl.kernel,
        out_type=jax.ShapeDtypeStruct((n, d), jnp.float32),
        mesh=mesh,
        scratch_types=[
            pltpu.VMEM((_CE,), jnp.int32),
            pltpu.VMEM((_CE,), jnp.int32),
            pltpu.VMEM((1, _CE), jnp.int32),
            pltpu.VMEM((_CE, d), jnp.float32),
            pltpu.VMEM((rc, d), jnp.float32),
            pltpu.SemaphoreType.DMA,
            pltpu.VMEM_SHARED((half + 8, d), jnp.float32),
        ],
        compiler_params=pltpu.CompilerParams(use_tc_tiling_on_sc=False),
    )
    def prop_k(z_hbm, src_hbm, dst_hbm, out_hbm,
               sbuf, dbuf, widx, rows, stage, sem, acc):
        c = lax.axis_index("c")
        s = lax.axis_index("s")
        base_node = c * half
        zeros16 = jnp.zeros((16,), jnp.float32)

        def zstage(r, carry):
            for v in range(d // 16):
                stage[r, pl.ds(v * 16, 16)] = zeros16
            return carry
        lax.fori_loop(0, rc, zstage, None)

        def zacc(k, carry):
            cc = s + k * _NS

            @pl.when(cc < nrc)
            def _():
                pltpu.sync_copy(stage, acc.at[pl.ds(cc * rc, rc)])
            return carry
        lax.fori_loop(0, kr, zacc, None)

        @pl.when(s == 0)
        def _():
            pltpu.sync_copy(stage.at[pl.ds(0, 8)], acc.at[pl.ds(half, 8)])
        plsc.subcore_barrier()

        ebase = s * ept

        def ebody(j, carry):
            off = ebase + j * _CE
            pltpu.sync_copy(src_hbm.at[pl.ds(off, _CE)], sbuf)
            pltpu.sync_copy(dst_hbm.at[pl.ds(off, _CE)], dbuf)
            for v in range(_CE // 16):
                dv = dbuf[pl.ds(v * 16, 16)]
                local = dv - base_node
                ok = (local >= 0) & (local < half)
                widx[0, pl.ds(v * 16, 16)] = jnp.where(ok, local, half)
            pltpu.async_copy(z_hbm.at[sbuf], rows, sem).wait()
            pltpu.sync_copy(rows, acc.at[widx.at[0]], add=True)
            return carry
        lax.fori_loop(0, nchunk, ebody, None)
        plsc.subcore_barrier()

        def dbody(k, carry):
            cc = s + k * _NS

            @pl.when(cc < nrc)
            def _():
                pltpu.sync_copy(acc.at[pl.ds(cc * rc, rc)], stage)
                pltpu.sync_copy(
                    stage, out_hbm.at[pl.ds(base_node + cc * rc, rc)])
            return carry
        lax.fori_loop(0, kr, dbody, None)

    return prop_k


def kernel(x, edge_index):
    n, d = x.shape
    e = edge_index.shape[1]
    src = edge_index[0]
    dst = edge_index[1]

    h, xt = _ew1(x)
    degc = _make_deg(n, e)(dst)
    dinv, z = _scale1(degc.reshape(n, 1), xt)
    prop = _make_prop(n, d, e)
    s1 = prop(z, src, dst)
    z2 = _mid(dinv, s1, z)
    s2 = prop(z2, src, dst)
    h2 = _fin(dinv, s2, z2)
    return (h, h2)


# speedup vs baseline: 24.9154x; 3.1715x over previous
import functools

import jax
import jax.numpy as jnp
from jax import lax
from jax.experimental import pallas as pl
from jax.experimental.pallas import tpu as pltpu
from jax.experimental.pallas import tpu_sc as plsc

_NC = 2
_NS = 16
_CE = 80
_EB = 2000
_NSLOT = 3


def _ew1_body(x_ref, h_ref, xt_ref):
    xt = jnp.tanh(x_ref[...])
    xt_ref[...] = xt
    h_ref[...] = jnp.sign(xt)


def _scale1_body(deg_ref, xt_ref, dinv_ref, z_ref):
    dinv = lax.rsqrt(deg_ref[...] + 1.0)
    dinv_ref[...] = dinv
    z_ref[...] = dinv * xt_ref[...]


def _mid_body(dinv_ref, s1_ref, z_ref, z2_ref):
    dinv = dinv_ref[...]
    z2_ref[...] = dinv * dinv * (s1_ref[...] + z_ref[...])


def _fin_body(dinv_ref, s2_ref, z2_ref, h2_ref):
    h2_ref[...] = dinv_ref[...] * (s2_ref[...] + z2_ref[...])


def _row_spec(br, d):
    return pl.BlockSpec((br, d), lambda i: (i, 0))


def _ew1(x):
    n, d = x.shape
    br = 2000
    return pl.pallas_call(
        _ew1_body,
        grid=(n // br,),
        in_specs=[_row_spec(br, d)],
        out_specs=[_row_spec(br, d), _row_spec(br, d)],
        out_shape=[jax.ShapeDtypeStruct((n, d), jnp.float32)] * 2,
    )(x)


def _scale1(degc, xt):
    n, d = xt.shape
    br = 2000
    return pl.pallas_call(
        _scale1_body,
        grid=(n // br,),
        in_specs=[_row_spec(br, 1), _row_spec(br, d)],
        out_specs=[_row_spec(br, 1), _row_spec(br, d)],
        out_shape=[jax.ShapeDtypeStruct((n, 1), jnp.float32),
                   jax.ShapeDtypeStruct((n, d), jnp.float32)],
    )(degc, xt)


def _mid(dinv, s1, z):
    n, d = z.shape
    br = 2000
    return pl.pallas_call(
        _mid_body,
        grid=(n // br,),
        in_specs=[_row_spec(br, 1), _row_spec(br, d), _row_spec(br, d)],
        out_specs=_row_spec(br, d),
        out_shape=jax.ShapeDtypeStruct((n, d), jnp.float32),
    )(dinv, s1, z)


def _fin(dinv, s2, z2):
    n, d = z2.shape
    br = 2000
    return pl.pallas_call(
        _fin_body,
        grid=(n // br,),
        in_specs=[_row_spec(br, 1), _row_spec(br, d), _row_spec(br, d)],
        out_specs=_row_spec(br, d),
        out_shape=jax.ShapeDtypeStruct((n, d), jnp.float32),
    )(dinv, s2, z2)


@functools.lru_cache(maxsize=None)
def _make_deg(n, e):
    half = n // _NC
    ept = e // _NS
    nblk = ept // _EB
    ncpb = _EB // _CE
    ntri = ncpb // _NSLOT
    npeel = ncpb - ntri * _NSLOT
    dc = 1000
    ndc = half // dc
    kd = -(-ndc // _NS)
    mesh = plsc.VectorSubcoreMesh(core_axis_name="c", subcore_axis_name="s")

    @functools.partial(
        pl.kernel,
        out_type=jax.ShapeDtypeStruct((n,), jnp.float32),
        mesh=mesh,
        scratch_types=[
            [pltpu.VMEM((_EB,), jnp.int32)] * 2,
            pltpu.VMEM((_NSLOT, _CE), jnp.int32),
            pltpu.VMEM((_CE,), jnp.float32),
            pltpu.VMEM((dc,), jnp.float32),
            [pltpu.SemaphoreType.DMA] * 2,
            [pltpu.SemaphoreType.DMA] * _NSLOT,
            pltpu.VMEM_SHARED((half + 16,), jnp.float32),
        ],
        compiler_params=pltpu.CompilerParams(use_tc_tiling_on_sc=False),
    )
    def deg_k(dst_hbm, out_hbm, dbig, widx, ones, stage, sis, scs, acc):
        c = lax.axis_index("c")
        s = lax.axis_index("s")
        base_node = c * half
        for v in range(_CE // 16):
            ones[pl.ds(v * 16, 16)] = jnp.ones((16,), jnp.float32)
        zeros16 = jnp.zeros((16,), jnp.float32)
        io16 = lax.iota(jnp.int32, 16)

        def zstage(v, carry):
            stage[pl.ds(v * 16, 16)] = zeros16
            return carry
        lax.fori_loop(0, dc // 16, zstage, None)

        def zacc(k, carry):
            cc = s + k * _NS

            @pl.when(cc < ndc)
            def _():
                pltpu.sync_copy(stage, acc.at[pl.ds(cc * dc, dc)])
            return carry
        lax.fori_loop(0, kd, zacc, None)

        @pl.when(s == 0)
        def _():
            pltpu.sync_copy(stage.at[pl.ds(0, 16)], acc.at[pl.ds(half, 16)])
        plsc.subcore_barrier()

        ebase = s * ept

        def fire_idx(blk, b):
            pltpu.async_copy(
                dst_hbm.at[pl.ds(ebase + blk * _EB, _EB)], dbig[b], sis[b])

        def wait_idx(b):
            pltpu.make_async_copy(
                dst_hbm.at[pl.ds(0, _EB)], dbig[b], sis[b]).wait()

        def wchunk(i, b, off):
            for v in range(_CE // 16):
                dv = dbig[b][pl.ds(off + v * 16, 16)]
                local = dv - base_node
                ok = (local >= 0) & (local < half)
                widx[i, pl.ds(v * 16, 16)] = jnp.where(ok, local, half + io16)

        def fire_sc(i):
            pltpu.async_copy(ones, acc.at[widx.at[i]], scs[i], add=True)

        def wait_sc(i):
            pltpu.make_async_copy(ones, acc.at[widx.at[i]], scs[i]).wait()

        def do_block(blk, b):
            wait_idx(b)

            @pl.when(blk + 1 < nblk)
            def _():
                fire_idx(blk + 1, 1 - b)

            def tri(q2, carry):
                for i in range(_NSLOT):
                    @pl.when(q2 > 0)
                    def _():
                        wait_sc(i)
                    wchunk(i, b, (q2 * _NSLOT + i) * _CE)
                    fire_sc(i)
                return carry
            lax.fori_loop(0, ntri, tri, None)
            for p in range(npeel):
                wait_sc(p)
                wchunk(p, b, (ntri * _NSLOT + p) * _CE)
                fire_sc(p)
            for i in range(npeel, _NSLOT):
                wait_sc(i)
            for p in range(npeel):
                wait_sc(p)

        fire_idx(0, 0)

        def bpair(bp, carry):
            do_block(2 * bp, 0)
            do_block(2 * bp + 1, 1)
            return carry
        lax.fori_loop(0, nblk // 2, bpair, None)
        if nblk % 2:
            do_block(nblk - 1, 0)
        plsc.subcore_barrier()

        def dbody(k, carry):
            cc = s + k * _NS

            @pl.when(cc < ndc)
            def _():
                pltpu.sync_copy(acc.at[pl.ds(cc * dc, dc)], stage)
                pltpu.sync_copy(
                    stage, out_hbm.at[pl.ds(base_node + cc * dc, dc)])
            return carry
        lax.fori_loop(0, kd, dbody, None)

    return deg_k


@functools.lru_cache(maxsize=None)
def _make_prop(n, d, e):
    half = n // _NC
    ept = e // _NS
    nblk = ept // _EB
    ncpb = _EB // _CE
    ntri = ncpb // _NSLOT
    npeel = ncpb - ntri * _NSLOT
    rc = 40
    nrc = half // rc
    kr = -(-nrc // _NS)
    mesh = plsc.VectorSubcoreMesh(core_axis_name="c", subcore_axis_name="s")

    @functools.partial(
        pl.kernel,
        out_type=jax.ShapeDtypeStruct((n, d), jnp.float32),
        mesh=mesh,
        scratch_types=[
            [pltpu.VMEM((_EB,), jnp.int32)] * 2,
            [pltpu.VMEM((_EB,), jnp.int32)] * 2,
            pltpu.VMEM((_NSLOT, _CE), jnp.int32),
            [pltpu.VMEM((_CE, d), jnp.float32)] * _NSLOT,
            pltpu.VMEM((rc, d), jnp.float32),
            [pltpu.SemaphoreType.DMA] * 2,
            [pltpu.SemaphoreType.DMA] * _NSLOT,
            [pltpu.SemaphoreType.DMA] * _NSLOT,
            pltpu.VMEM_SHARED((half + 32, d), jnp.float32),
        ],
        compiler_params=pltpu.CompilerParams(use_tc_tiling_on_sc=False),
    )
    def prop_k(z_hbm, src_hbm, dst_hbm, out_hbm,
               sbig, dbig, widx, rows, stage, sis, sgs, scs, acc):
        c = lax.axis_index("c")
        s = lax.axis_index("s")
        base_node = c * half
        zeros16 = jnp.zeros((16,), jnp.float32)
        io16 = lax.iota(jnp.int32, 16)

        def zstage(r, carry):
            for v in range(d // 16):
                stage[r, pl.ds(v * 16, 16)] = zeros16
            return carry
        lax.fori_loop(0, rc, zstage, None)

        def zacc(k, carry):
            cc = s + k * _NS

            @pl.when(cc < nrc)
            def _():
                pltpu.sync_copy(stage, acc.at[pl.ds(cc * rc, rc)])
            return carry
        lax.fori_loop(0, kr, zacc, None)

        @pl.when(s == 0)
        def _():
            pltpu.sync_copy(stage.at[pl.ds(0, 32)], acc.at[pl.ds(half, 32)])
        plsc.subcore_barrier()

        ebase = s * ept

        def fire_idx(blk, b):
            off = ebase + blk * _EB
            pltpu.async_copy(src_hbm.at[pl.ds(off, _EB)], sbig[b], sis[b])
            pltpu.async_copy(dst_hbm.at[pl.ds(off, _EB)], dbig[b], sis[b])

        def wait_idx(b):
            pltpu.make_async_copy(
                src_hbm.at[pl.ds(0, _EB)], sbig[b], sis[b]).wait()
            pltpu.make_async_copy(
                dst_hbm.at[pl.ds(0, _EB)], dbig[b], sis[b]).wait()

        def wchunk(i, b, off):
            for v in range(_CE // 16):
                dv = dbig[b][pl.ds(off + v * 16, 16)]
                local = dv - base_node
                ok = (local >= 0) & (local < half)
                trash = half + io16 + (16 * (v % 2))
                widx[i, pl.ds(v * 16, 16)] = jnp.where(ok, local, trash)

        def fire_gather(i, b, off):
            pltpu.async_copy(
                z_hbm.at[sbig[b].at[pl.ds(off, _CE)]], rows[i], sgs[i])

        def wait_g(i):
            pltpu.make_async_copy(
                z_hbm.at[sbig[0].at[pl.ds(0, _CE)]], rows[i], sgs[i]).wait()

        def fire_scatter(i):
            pltpu.async_copy(rows[i], acc.at[widx.at[i]], scs[i], add=True)

        def wait_sc(i):
            pltpu.make_async_copy(rows[i], acc.at[widx.at[i]], scs[i]).wait()

        def do_block(blk, b):
            wait_idx(b)

            @pl.when(blk + 1 < nblk)
            def _():
                fire_idx(blk + 1, 1 - b)

            def tri(q2, carry):
                for i in range(_NSLOT):
                    @pl.when(q2 > 0)
                    def _():
                        wait_sc(i)
                    off = (q2 * _NSLOT + i) * _CE
                    wchunk(i, b, off)
                    fire_gather(i, b, off)
                    pi = (i + _NSLOT - 1) % _NSLOT
                    if i == 0:
                        @pl.when(q2 > 0)
                        def _():
                            wait_g(pi)
                            fire_scatter(pi)
                    else:
                        wait_g(pi)
                        fire_scatter(pi)
                return carry
            lax.fori_loop(0, ntri, tri, None)
            for p in range(npeel):
                wait_sc(p)
                off = (ntri * _NSLOT + p) * _CE
                wchunk(p, b, off)
                fire_gather(p, b, off)
                pi = (p + _NSLOT - 1) % _NSLOT
                wait_g(pi)
                fire_scatter(pi)
            last = (npeel + _NSLOT - 1) % _NSLOT
            wait_g(last)
            fire_scatter(last)
            for i in range(npeel, _NSLOT):
                wait_sc(i)
            for p in range(npeel):
                wait_sc(p)

        fire_idx(0, 0)

        def bpair(bp, carry):
            do_block(2 * bp, 0)
            do_block(2 * bp + 1, 1)
            return carry
        lax.fori_loop(0, nblk // 2, bpair, None)
        if nblk % 2:
            do_block(nblk - 1, 0)
        plsc.subcore_barrier()

        def dbody(k, carry):
            cc = s + k * _NS

            @pl.when(cc < nrc)
            def _():
                pltpu.sync_copy(acc.at[pl.ds(cc * rc, rc)], stage)
                pltpu.sync_copy(
                    stage, out_hbm.at[pl.ds(base_node + cc * rc, rc)])
            return carry
        lax.fori_loop(0, kr, dbody, None)

    return prop_k


def kernel(x, edge_index):
    n, d = x.shape
    e = edge_index.shape[1]
    src = edge_index[0]
    dst = edge_index[1]

    h, xt = _ew1(x)
    degc = _make_deg(n, e)(dst)
    dinv, z = _scale1(degc.reshape(n, 1), xt)
    prop = _make_prop(n, d, e)
    s1 = prop(z, src, dst)
    z2 = _mid(dinv, s1, z)
    s2 = prop(z2, src, dst)
    h2 = _fin(dinv, s2, z2)
    return (h, h2)


# speedup vs baseline: 26.8590x; 1.0780x over previous
import functools

import jax
import jax.numpy as jnp
from jax import lax
from jax.experimental import pallas as pl
from jax.experimental.pallas import tpu as pltpu
from jax.experimental.pallas import tpu_sc as plsc

_NC = 2
_NS = 16
_CE = 80
_EB = 2000
_NSLOT = 3


def _ew1_body(x_ref, h_ref, xt_ref):
    xt = jnp.tanh(x_ref[...])
    xt_ref[...] = xt
    h_ref[...] = jnp.sign(xt)


def _scale1_body(deg_ref, xt_ref, dinv_ref, z_ref):
    dinv = lax.rsqrt(deg_ref[...] + 1.0)
    dinv_ref[...] = dinv
    z_ref[...] = dinv * xt_ref[...]


def _mid_body(dinv_ref, s1_ref, z_ref, z2_ref):
    dinv = dinv_ref[...]
    z2_ref[...] = dinv * dinv * (s1_ref[...] + z_ref[...])


def _fin_body(dinv_ref, s2_ref, z2_ref, h2_ref):
    h2_ref[...] = dinv_ref[...] * (s2_ref[...] + z2_ref[...])


def _row_spec(br, d):
    return pl.BlockSpec((br, d), lambda i: (i, 0))


def _ew1(x):
    n, d = x.shape
    br = 2000
    return pl.pallas_call(
        _ew1_body,
        grid=(n // br,),
        in_specs=[_row_spec(br, d)],
        out_specs=[_row_spec(br, d), _row_spec(br, d)],
        out_shape=[jax.ShapeDtypeStruct((n, d), jnp.float32)] * 2,
    )(x)


def _scale1(degc, xt):
    n, d = xt.shape
    br = 2000
    return pl.pallas_call(
        _scale1_body,
        grid=(n // br,),
        in_specs=[_row_spec(br, 1), _row_spec(br, d)],
        out_specs=[_row_spec(br, 1), _row_spec(br, d)],
        out_shape=[jax.ShapeDtypeStruct((n, 1), jnp.float32),
                   jax.ShapeDtypeStruct((n, d), jnp.float32)],
    )(degc, xt)


def _mid(dinv, s1, z):
    n, d = z.shape
    br = 2000
    return pl.pallas_call(
        _mid_body,
        grid=(n // br,),
        in_specs=[_row_spec(br, 1), _row_spec(br, d), _row_spec(br, d)],
        out_specs=_row_spec(br, d),
        out_shape=jax.ShapeDtypeStruct((n, d), jnp.float32),
    )(dinv, s1, z)


def _fin(dinv, s2, z2):
    n, d = z2.shape
    br = 2000
    return pl.pallas_call(
        _fin_body,
        grid=(n // br,),
        in_specs=[_row_spec(br, 1), _row_spec(br, d), _row_spec(br, d)],
        out_specs=_row_spec(br, d),
        out_shape=jax.ShapeDtypeStruct((n, d), jnp.float32),
    )(dinv, s2, z2)


@functools.lru_cache(maxsize=None)
def _make_deg(n, e):
    half = n // _NC
    ept = e // _NS
    nblk = ept // _EB
    ncpb = _EB // _CE
    ntri = ncpb // _NSLOT
    npeel = ncpb - ntri * _NSLOT
    dc = 1000
    ndc = half // dc
    kd = -(-ndc // _NS)
    mesh = plsc.VectorSubcoreMesh(core_axis_name="c", subcore_axis_name="s")

    @functools.partial(
        pl.kernel,
        out_type=jax.ShapeDtypeStruct((n,), jnp.float32),
        mesh=mesh,
        scratch_types=[
            [pltpu.VMEM((_EB,), jnp.int32)] * 2,
            pltpu.VMEM((_NSLOT, _CE), jnp.int32),
            pltpu.VMEM((_CE,), jnp.float32),
            pltpu.VMEM((dc,), jnp.float32),
            [pltpu.SemaphoreType.DMA] * 2,
            [pltpu.SemaphoreType.DMA] * _NSLOT,
            pltpu.VMEM_SHARED((half,), jnp.float32),
        ],
        compiler_params=pltpu.CompilerParams(use_tc_tiling_on_sc=False),
    )
    def deg_k(dst_hbm, out_hbm, dbig, widx, ones, stage, sis, scs, acc):
        c = lax.axis_index("c")
        s = lax.axis_index("s")
        base_node = c * half
        for v in range(_CE // 16):
            ones[pl.ds(v * 16, 16)] = jnp.ones((16,), jnp.float32)
        zeros16 = jnp.zeros((16,), jnp.float32)
        io16 = lax.iota(jnp.int32, 16)

        def zstage(v, carry):
            stage[pl.ds(v * 16, 16)] = zeros16
            return carry
        lax.fori_loop(0, dc // 16, zstage, None)

        def zacc(k, carry):
            cc = s + k * _NS

            @pl.when(cc < ndc)
            def _():
                pltpu.sync_copy(stage, acc.at[pl.ds(cc * dc, dc)])
            return carry
        lax.fori_loop(0, kd, zacc, None)

        plsc.subcore_barrier()

        ebase = s * ept

        def fire_idx(blk, b):
            pltpu.async_copy(
                dst_hbm.at[pl.ds(ebase + blk * _EB, _EB)], dbig[b], sis[b])

        def wait_idx(b):
            pltpu.make_async_copy(
                dst_hbm.at[pl.ds(0, _EB)], dbig[b], sis[b]).wait()

        def wchunk(i, b, off):
            for v in range(_CE // 16):
                dv = dbig[b][pl.ds(off + v * 16, 16)]
                local = dv - base_node
                ok = (local >= 0) & (local < half)
                widx[i, pl.ds(v * 16, 16)] = jnp.where(ok, local, -1)

        def _sc_dst(i):
            return acc.at[plsc.Indices(widx.at[i], ignored_value=-1)]

        def fire_sc(i):
            pltpu.async_copy(ones, _sc_dst(i), scs[i], add=True)

        def wait_sc(i):
            pltpu.make_async_copy(ones, _sc_dst(i), scs[i]).wait()

        def do_block(blk, b):
            wait_idx(b)

            @pl.when(blk + 1 < nblk)
            def _():
                fire_idx(blk + 1, 1 - b)

            def tri(q2, carry):
                for i in range(_NSLOT):
                    @pl.when(q2 > 0)
                    def _():
                        wait_sc(i)
                    wchunk(i, b, (q2 * _NSLOT + i) * _CE)
                    fire_sc(i)
                return carry
            lax.fori_loop(0, ntri, tri, None)
            for p in range(npeel):
                wait_sc(p)
                wchunk(p, b, (ntri * _NSLOT + p) * _CE)
                fire_sc(p)
            for i in range(npeel, _NSLOT):
                wait_sc(i)
            for p in range(npeel):
                wait_sc(p)

        fire_idx(0, 0)

        def bpair(bp, carry):
            do_block(2 * bp, 0)
            do_block(2 * bp + 1, 1)
            return carry
        lax.fori_loop(0, nblk // 2, bpair, None)
        if nblk % 2:
            do_block(nblk - 1, 0)
        plsc.subcore_barrier()

        def dbody(k, carry):
            cc = s + k * _NS

            @pl.when(cc < ndc)
            def _():
                pltpu.sync_copy(acc.at[pl.ds(cc * dc, dc)], stage)
                pltpu.sync_copy(
                    stage, out_hbm.at[pl.ds(base_node + cc * dc, dc)])
            return carry
        lax.fori_loop(0, kd, dbody, None)

    return deg_k


@functools.lru_cache(maxsize=None)
def _make_prop(n, d, e):
    half = n // _NC
    ept = e // _NS
    nblk = ept // _EB
    ncpb = _EB // _CE
    ntri = ncpb // _NSLOT
    npeel = ncpb - ntri * _NSLOT
    rc = 40
    nrc = half // rc
    kr = -(-nrc // _NS)
    mesh = plsc.VectorSubcoreMesh(core_axis_name="c", subcore_axis_name="s")

    @functools.partial(
        pl.kernel,
        out_type=jax.ShapeDtypeStruct((n, d), jnp.float32),
        mesh=mesh,
        scratch_types=[
            [pltpu.VMEM((_EB,), jnp.int32)] * 2,
            [pltpu.VMEM((_EB,), jnp.int32)] * 2,
            pltpu.VMEM((_NSLOT, _CE), jnp.int32),
            pltpu.VMEM((_NSLOT, _CE), jnp.int32),
            [pltpu.VMEM((_CE, d), jnp.float32)] * _NSLOT,
            pltpu.VMEM((rc, d), jnp.float32),
            [pltpu.SemaphoreType.DMA] * 2,
            [pltpu.SemaphoreType.DMA] * _NSLOT,
            [pltpu.SemaphoreType.DMA] * _NSLOT,
            pltpu.VMEM_SHARED((half, d), jnp.float32),
        ],
        compiler_params=pltpu.CompilerParams(use_tc_tiling_on_sc=False),
    )
    def prop_k(z_hbm, src_hbm, dst_hbm, out_hbm,
               sbig, dbig, widx, gidx, rows, stage, sis, sgs, scs, acc):
        c = lax.axis_index("c")
        s = lax.axis_index("s")
        base_node = c * half
        zeros16 = jnp.zeros((16,), jnp.float32)
        io16 = lax.iota(jnp.int32, 16)

        def zstage(r, carry):
            for v in range(d // 16):
                stage[r, pl.ds(v * 16, 16)] = zeros16
            return carry
        lax.fori_loop(0, rc, zstage, None)

        def zacc(k, carry):
            cc = s + k * _NS

            @pl.when(cc < nrc)
            def _():
                pltpu.sync_copy(stage, acc.at[pl.ds(cc * rc, rc)])
            return carry
        lax.fori_loop(0, kr, zacc, None)

        plsc.subcore_barrier()

        ebase = s * ept

        def fire_idx(blk, b):
            off = ebase + blk * _EB
            pltpu.async_copy(src_hbm.at[pl.ds(off, _EB)], sbig[b], sis[b])
            pltpu.async_copy(dst_hbm.at[pl.ds(off, _EB)], dbig[b], sis[b])

        def wait_idx(b):
            pltpu.make_async_copy(
                src_hbm.at[pl.ds(0, _EB)], sbig[b], sis[b]).wait()
            pltpu.make_async_copy(
                dst_hbm.at[pl.ds(0, _EB)], dbig[b], sis[b]).wait()

        def wchunk(i, b, off):
            for v in range(_CE // 16):
                dv = dbig[b][pl.ds(off + v * 16, 16)]
                sv = sbig[b][pl.ds(off + v * 16, 16)]
                local = dv - base_node
                ok = (local >= 0) & (local < half)
                widx[i, pl.ds(v * 16, 16)] = jnp.where(ok, local, -1)
                gidx[i, pl.ds(v * 16, 16)] = jnp.where(ok, sv, -1)

        def _g_src(i):
            return z_hbm.at[plsc.Indices(gidx.at[i], ignored_value=-1)]

        def _sc_dst(i):
            return acc.at[plsc.Indices(widx.at[i], ignored_value=-1)]

        def fire_gather(i, b, off):
            pltpu.async_copy(_g_src(i), rows[i], sgs[i])

        def wait_g(i):
            pltpu.make_async_copy(_g_src(i), rows[i], sgs[i]).wait()

        def fire_scatter(i):
            pltpu.async_copy(rows[i], _sc_dst(i), scs[i], add=True)

        def wait_sc(i):
            pltpu.make_async_copy(rows[i], _sc_dst(i), scs[i]).wait()

        def do_block(blk, b):
            wait_idx(b)

            @pl.when(blk + 1 < nblk)
            def _():
                fire_idx(blk + 1, 1 - b)

            def tri(q2, carry):
                for i in range(_NSLOT):
                    @pl.when(q2 > 0)
                    def _():
                        wait_sc(i)
                    off = (q2 * _NSLOT + i) * _CE
                    wchunk(i, b, off)
                    fire_gather(i, b, off)
                    pi = (i + _NSLOT - 1) % _NSLOT
                    if i == 0:
                        @pl.when(q2 > 0)
                        def _():
                            wait_g(pi)
                            fire_scatter(pi)
                    else:
                        wait_g(pi)
                        fire_scatter(pi)
                return carry
            lax.fori_loop(0, ntri, tri, None)
            for p in range(npeel):
                wait_sc(p)
                off = (ntri * _NSLOT + p) * _CE
                wchunk(p, b, off)
                fire_gather(p, b, off)
                pi = (p + _NSLOT - 1) % _NSLOT
                wait_g(pi)
                fire_scatter(pi)
            last = (npeel + _NSLOT - 1) % _NSLOT
            wait_g(last)
            fire_scatter(last)
            for i in range(npeel, _NSLOT):
                wait_sc(i)
            for p in range(npeel):
                wait_sc(p)

        fire_idx(0, 0)

        def bpair(bp, carry):
            do_block(2 * bp, 0)
            do_block(2 * bp + 1, 1)
            return carry
        lax.fori_loop(0, nblk // 2, bpair, None)
        if nblk % 2:
            do_block(nblk - 1, 0)
        plsc.subcore_barrier()

        def dbody(k, carry):
            cc = s + k * _NS

            @pl.when(cc < nrc)
            def _():
                pltpu.sync_copy(acc.at[pl.ds(cc * rc, rc)], stage)
                pltpu.sync_copy(
                    stage, out_hbm.at[pl.ds(base_node + cc * rc, rc)])
            return carry
        lax.fori_loop(0, kr, dbody, None)

    return prop_k


def kernel(x, edge_index):
    n, d = x.shape
    e = edge_index.shape[1]
    src = edge_index[0]
    dst = edge_index[1]

    h, xt = _ew1(x)
    degc = _make_deg(n, e)(dst)
    dinv, z = _scale1(degc.reshape(n, 1), xt)
    prop = _make_prop(n, d, e)
    s1 = prop(z, src, dst)
    z2 = _mid(dinv, s1, z)
    s2 = prop(z2, src, dst)
    h2 = _fin(dinv, s2, z2)
    return (h, h2)
